# Initial kernel scaffold; baseline (speedup 1.0000x reference)
import jax
import jax.numpy as jnp
from jax.experimental import pallas as pl


def kernel(data, params):
    raise NotImplementedError("write your pallas kernel here")


# speedup vs baseline: 7.9230x; 7.9230x over previous
import functools

import jax
import jax.numpy as jnp
from jax import lax
from jax.experimental import pallas as pl
from jax.experimental.pallas import tpu as pltpu
from jax.experimental.pallas import tpu_sc as plsc

_F32 = jnp.float32
_HI = jax.lax.Precision.HIGHEST


def _dot(a, b, precision=_HI):
    return jax.lax.dot_general(
        a, b, (((1,), (0,)), ((), ())),
        precision=precision, preferred_element_type=_F32)


def _sqdist(nmat, xyzt):
    x = xyzt[0:1, :]
    y = xyzt[1:2, :]
    z = xyzt[2:3, :]
    sq = x * x + y * y + z * z
    nx = nmat[:, 0:1]
    ny = nmat[:, 1:2]
    nz = nmat[:, 2:3]
    nsq = nx * nx + ny * ny + nz * nz
    inner = _dot(nmat, xyzt, precision=jax.lax.Precision.DEFAULT)
    return nsq + sq - 2.0 * inner


def _fps_kernel(npoint, n, x_ref, y_ref, z_ref, ox_ref, oy_ref, oz_ref):
    rows, cols = x_ref.shape
    iota = (lax.broadcasted_iota(jnp.int32, (rows, cols), 0) * cols
            + lax.broadcasted_iota(jnp.int32, (rows, cols), 1))
    x = x_ref[...]
    y = y_ref[...]
    z = z_ref[...]

    def step(s, carry):
        dist, far = carry
        onehot = iota == far
        cx = jnp.sum(jnp.where(onehot, x, 0.0))
        cy = jnp.sum(jnp.where(onehot, y, 0.0))
        cz = jnp.sum(jnp.where(onehot, z, 0.0))
        ox_ref[pl.ds(s, 1), :] = cx * jnp.ones((1, 1), _F32)
        oy_ref[pl.ds(s, 1), :] = cy * jnp.ones((1, 1), _F32)
        oz_ref[pl.ds(s, 1), :] = cz * jnp.ones((1, 1), _F32)
        dx = x - cx
        dy = y - cy
        dz = z - cz
        d = (dx * dx + dy * dy) + dz * dz
        dist = jnp.minimum(dist, d)
        m = jnp.max(dist)
        far_new = jnp.min(jnp.where(dist == m, iota, n))
        return dist, far_new

    init = (jnp.full((rows, cols), 1e10, _F32), jnp.int32(0))
    lax.fori_loop(0, npoint, step, init)


def _fps(x2d, y2d, z2d, npoint, n):
    out = jax.ShapeDtypeStruct((npoint, 1), _F32)
    f = functools.partial(_fps_kernel, npoint, n)
    return pl.pallas_call(
        f, out_shape=(out, out, out))(x2d, y2d, z2d)


def _bq_kernel(n, specs, nmat_ref, xyzt_ref, g1_ref, g2_ref):
    d = _sqdist(nmat_ref[...], xyzt_ref[...])
    mb = d.shape[0]
    iota = lax.broadcasted_iota(jnp.int32, (mb, n), 1)
    for (r2, k, ref) in ((specs[0][0], specs[0][1], g1_ref),
                         (specs[1][0], specs[1][1], g2_ref)):
        masked = jnp.where(d > r2, n, iota)
        cols = []
        for _ in range(k):
            m = jnp.min(masked, axis=1, keepdims=True)
            cols.append(m)
            masked = jnp.where(masked == m, n, masked)
        gi = jnp.concatenate(cols, axis=1)
        gi = jnp.where(gi == n, cols[0], gi)
        ref[...] = gi


def _ball_query(nmat, xyzt, specs, m, n):
    mb = min(m, 64)
    grid = m // mb
    k1 = specs[0][1]
    k2 = specs[1][1]
    f = functools.partial(_bq_kernel, n, specs)
    return pl.pallas_call(
        f,
        grid=(grid,),
        in_specs=[
            pl.BlockSpec((mb, 3), lambda i: (i, 0)),
            pl.BlockSpec((3, n), lambda i: (0, 0)),
        ],
        out_specs=(
            pl.BlockSpec((mb, k1), lambda i: (i, 0)),
            pl.BlockSpec((mb, k2), lambda i: (i, 0)),
        ),
        out_shape=(
            jax.ShapeDtypeStruct((m, k1), jnp.int32),
            jax.ShapeDtypeStruct((m, k2), jnp.int32),
        ),
    )(nmat, xyzt)


def _sc_gather(table, idx):
    v, d = table.shape
    b = idx.shape[0]
    info = plsc.get_sparse_core_info()
    nw = info.num_cores * info.num_subcores
    nc = info.num_cores
    ch = 128
    granule = nw * ch
    b_pad = -(-b // granule) * granule
    if b_pad != b:
        pad = jnp.arange(b_pad - b, dtype=jnp.int32) % v
        idx = jnp.concatenate([idx, pad])
    b_per_w = b_pad // nw
    nch = b_per_w // ch
    idx2 = idx.reshape(nw * nch, ch)
    mesh = plsc.VectorSubcoreMesh(core_axis_name="c", subcore_axis_name="s")

    def body(table_hbm, idx_hbm, out_hbm, *scratch):
        idx_bufs = scratch[:nch]
        rows_v = scratch[nch]
        sem = scratch[nch + 1]
        wid = lax.axis_index("s") * nc + lax.axis_index("c")
        for j in range(nch):
            pltpu.sync_copy(idx_hbm.at[wid * nch + j], idx_bufs[j])
            pltpu.async_copy(table_hbm.at[idx_bufs[j]], rows_v, sem).wait()
            pltpu.sync_copy(
                rows_v, out_hbm.at[pl.ds(wid * b_per_w + j * ch, ch)])

    body.__name__ = "sc_gather_v%d_b%d_d%d" % (v, b_pad, d)
    k = pl.kernel(
        body, mesh=mesh,
        out_type=jax.ShapeDtypeStruct((b_pad, d), _F32),
        scratch_types=(
            [pltpu.VMEM((ch,), jnp.int32) for _ in range(nch)]
            + [pltpu.VMEM((ch, d), _F32), pltpu.SemaphoreType.DMA]))
    out = k(table, idx2)
    return out if b_pad == b else out[:b]


def _bn_relu(h, g, be):
    mu = jnp.mean(h, axis=0, keepdims=True)
    var = jnp.mean((h - mu) ** 2, axis=0, keepdims=True)
    h = (h - mu) / jnp.sqrt(var + 1e-5) * g + be
    return jnp.maximum(h, 0.0)


def _sa_mlp_kernel(m, k, n_layers, g_ref, nx_ref, w1x_ref, *refs):
    wrefs = refs[:4 * n_layers]
    out_ref = refs[-1]
    g = g_ref[...]
    h = _dot(g, wrefs[0][...]) + wrefs[1][...]
    sh = _dot(nx_ref[...], w1x_ref[...])
    c1 = sh.shape[1]
    she = jnp.reshape(
        jnp.broadcast_to(sh[:, None, :], (m, k, c1)), (m * k, c1))
    h = h - she
    h = _bn_relu(h, wrefs[2][...], wrefs[3][...])
    for li in range(1, n_layers):
        w, bb, gg, be = wrefs[4 * li:4 * li + 4]
        h = _dot(h, w[...]) + bb[...]
        h = _bn_relu(h, gg[...], be[...])
    hm = jnp.reshape(h, (m, k, h.shape[1]))
    out_ref[...] = jnp.max(hm, axis=1)


def _sa_mlp(gathered, nxyz3, layers, f_dim, m, k):
    dpad = gathered.shape[1]
    n_layers = len(layers)
    args = [gathered, nxyz3]
    w1 = layers[0][0]
    w1pad = jnp.zeros((dpad, w1.shape[1]), _F32).at[:f_dim + 3].set(w1)
    w1x = w1[f_dim:f_dim + 3]
    args.append(w1x)
    for li, (w, bb, gg, be) in enumerate(layers):
        args += [w1pad if li == 0 else w,
                 bb.reshape(1, -1), gg.reshape(1, -1), be.reshape(1, -1)]
    cout = layers[-1][0].shape[1]
    f = functools.partial(_sa_mlp_kernel, m, k, n_layers)
    return pl.pallas_call(
        f, out_shape=jax.ShapeDtypeStruct((m, cout), _F32))(*args)


def _fp_interp_kernel(m2, nmat_ref, xyzt_ref, p2_ref, out_ref):
    d = _sqdist(nmat_ref[...], xyzt_ref[...])
    mb = d.shape[0]
    iota = lax.broadcasted_iota(jnp.int32, (mb, m2), 1)
    ms = []
    ims = []
    for _ in range(3):
        mval = jnp.min(d, axis=1, keepdims=True)
        im = jnp.min(jnp.where(d == mval, iota, m2), axis=1, keepdims=True)
        d = jnp.where(iota == im, 1e30, d)
        ms.append(mval)
        ims.append(im)
    recips = [1.0 / (mv + 1e-8) for mv in ms]
    wsum = recips[0] + recips[1] + recips[2]
    wmat = jnp.zeros((mb, m2), _F32)
    for rv, im in zip(recips, ims):
        wmat = wmat + jnp.where(iota == im, rv / wsum, 0.0)
    out_ref[...] = _dot(wmat, p2_ref[...])


def _fp_interp(nmat1, xyzt2, p2, m1, m2):
    c2 = p2.shape[1]
    mb = min(m1, 2048)
    grid = m1 // mb
    f = functools.partial(_fp_interp_kernel, m2)
    return pl.pallas_call(
        f,
        grid=(grid,),
        in_specs=[
            pl.BlockSpec((mb, 3), lambda i: (i, 0)),
            pl.BlockSpec((3, m2), lambda i: (0, 0)),
            pl.BlockSpec((m2, c2), lambda i: (0, 0)),
        ],
        out_specs=pl.BlockSpec((mb, c2), lambda i: (i, 0)),
        out_shape=jax.ShapeDtypeStruct((m1, c2), _F32),
    )(nmat1, xyzt2, p2)


def _fp_mlp_kernel(n_layers, has_p1, *refs):
    if has_p1:
        p1_ref, i_ref = refs[0], refs[1]
        wrefs = refs[2:2 + 4 * n_layers]
        h = jnp.concatenate([p1_ref[...], i_ref[...]], axis=1)
    else:
        i_ref = refs[0]
        wrefs = refs[1:1 + 4 * n_layers]
        h = i_ref[...]
    out_ref = refs[-1]
    for li in range(n_layers):
        w, bb, gg, be = wrefs[4 * li:4 * li + 4]
        h = _dot(h, w[...]) + bb[...]
        h = _bn_relu(h, gg[...], be[...])
    out_ref[...] = h


def _fp_mlp(p1, interp, layers):
    n_layers = len(layers)
    args = [] if p1 is None else [p1]
    args.append(interp)
    for (w, bb, gg, be) in layers:
        args += [w, bb.reshape(1, -1), gg.reshape(1, -1), be.reshape(1, -1)]
    m1 = interp.shape[0]
    cout = layers[-1][0].shape[1]
    f = functools.partial(_fp_mlp_kernel, n_layers, p1 is not None)
    return pl.pallas_call(
        f, out_shape=jax.ShapeDtypeStruct((m1, cout), _F32))(*args)


def _pad_cols(a, dpad):
    d = a.shape[1]
    if d == dpad:
        return a
    return jnp.concatenate(
        [a, jnp.zeros((a.shape[0], dpad - d), _F32)], axis=1)


def _sa_level(xyz, points, npoint, radii, nsamples, branches):
    n = xyz.shape[0]
    use_sc = n >= 1024
    f_dim = points.shape[1]
    rows = n // 8
    x2d = xyz[:, 0].reshape(8, rows)
    y2d = xyz[:, 1].reshape(8, rows)
    z2d = xyz[:, 2].reshape(8, rows)
    ox, oy, oz = _fps(x2d, y2d, z2d, npoint, n)
    nxyz3 = jnp.concatenate([ox, oy, oz], axis=1)

    specs = ((radii[0] ** 2, nsamples[0]), (radii[1] ** 2, nsamples[1]))
    gi1, gi2 = _ball_query(nxyz3, xyz.T, specs, npoint, n)

    dpad = -(-(f_dim + 3) // 128) * 128
    table = _pad_cols(jnp.concatenate([points, xyz], axis=1), dpad)

    outs = []
    for gi, k, layers in ((gi1, nsamples[0], branches[0]),
                          (gi2, nsamples[1], branches[1])):
        if use_sc:
            g = _sc_gather(table, gi.reshape(-1))
        else:
            g = jnp.take(table, gi.reshape(-1), axis=0)
        outs.append(_sa_mlp(g, nxyz3, layers, f_dim, npoint, k))
    return nxyz3, jnp.concatenate(outs, axis=1)


def _fp_level(xyz1, xyz2, points1, points2, layers):
    m1 = xyz1.shape[0]
    m2 = xyz2.shape[0]
    interp = _fp_interp(xyz1, xyz2.T, points2, m1, m2)
    return _fp_mlp(points1, interp, layers)


def kernel(data, params):
    xyz0 = data[:, :3]
    p0 = data

    l1x, l1p = _sa_level(xyz0, p0, 512, [0.05, 0.1], [16, 32], params["sa1"])
    l2x, l2p = _sa_level(l1x, l1p, 256, [0.1, 0.2], [16, 32], params["sa2"])
    l3x, l3p = _sa_level(l2x, l2p, 64, [0.2, 0.4], [16, 32], params["sa3"])
    l4x, l4p = _sa_level(l3x, l3p, 16, [0.4, 0.8], [16, 32], params["sa4"])

    l3p = _fp_level(l3x, l4x, l3p, l4p, params["fp4"])
    l2p = _fp_level(l2x, l3x, l2p, l3p, params["fp3"])
    l1p = _fp_level(l1x, l2x, l1p, l2p, params["fp2"])
    l0p = _fp_level(xyz0, l1x, None, l1p, params["fp1"])
    return l0p
